# baseline (speedup 1.0000x reference)
import functools

import jax
import jax.numpy as jnp
import numpy as np
from jax import lax
from jax.experimental import pallas as pl
from jax.experimental.pallas import tpu as pltpu
from jax.experimental.pallas import tpu_sc as plsc

NQ = 300
NPATCH = 256
NPAD = 304
NCHUNK = NPAD // 16
KPAD = 112
KCHUNK = KPAD // 16
PRE = 100
KEEP = 20
OUTW = 160
PER_W = 8

_LANE = np.arange(16, dtype=np.int32)


def _nms_body(s_hbm, cx_hbm, cy_hbm, w_hbm, h_hbm, swsh_hbm, out_hbm,
              svbuf, cxbuf, cybuf, wbuf, hbuf, swshv,
              cs_r, ci_r, ss_r, sx1_r, sy1_r, sx2_r, sy2_r, ar_r,
              sup_r, pu_r, pv_r, stage_r):
    ncores = 2
    wid = lax.axis_index("s") * ncores + lax.axis_index("c")
    base = wid * PER_W

    pltpu.sync_copy(s_hbm.at[pl.ds(wid * (PER_W * NPAD), PER_W * NPAD)], svbuf)
    pltpu.sync_copy(cx_hbm.at[pl.ds(wid * (PER_W * NPAD), PER_W * NPAD)], cxbuf)
    pltpu.sync_copy(cy_hbm.at[pl.ds(wid * (PER_W * NPAD), PER_W * NPAD)], cybuf)
    pltpu.sync_copy(w_hbm.at[pl.ds(wid * (PER_W * NPAD), PER_W * NPAD)], wbuf)
    pltpu.sync_copy(h_hbm.at[pl.ds(wid * (PER_W * NPAD), PER_W * NPAD)], hbuf)
    pltpu.sync_copy(swsh_hbm.at[pl.ds(wid * 16, 16)], swshv)

    lane = lax.iota(jnp.int32, 16)
    zeros_i = jnp.full((16,), 0, jnp.int32)

    def splat(ref, i):
        return plsc.load_gather(ref, [jnp.full((16,), i, jnp.int32)])

    def vsplat(vec, l):
        return vec.at[jnp.full((16,), l, jnp.int32)].get(
            mode="promise_in_bounds")

    def patch_body(k, _):
        off = k * NPAD
        swv = splat(swshv, k)
        shv = splat(swshv, k + 8)

        svs = [plsc.load_gather(svbuf, [lane + (off + 16 * c)])
               for c in range(NCHUNK)]

        def bit_body(t, kbits):
            bitv = jnp.full((16,), lax.shift_left(jnp.int32(1), 29 - t),
                            jnp.int32)
            trial = jnp.bitwise_or(kbits, bitv)
            tv = plsc.bitcast(trial, jnp.float32)
            cnt = zeros_i
            for c in range(NCHUNK):
                cnt = cnt + plsc.all_reduce_population_count(svs[c] >= tv)
            return jnp.where(cnt >= PRE, trial, kbits)

        kbits = lax.fori_loop(0, 30, bit_body, zeros_i)
        thv = plsc.bitcast(kbits, jnp.float32)

        gcnt = zeros_i
        for c in range(NCHUNK):
            gcnt = gcnt + plsc.all_reduce_population_count(svs[c] > thv)
        tie_budget = PRE - gcnt

        ci_r[pl.ds(96, 16)] = lane + NPAD
        cs_r[pl.ds(96, 16)] = jnp.full((16,), -1.0, jnp.float32)
        nsel = zeros_i
        eqrun = zeros_i
        for c in range(NCHUNK):
            sv = svs[c]
            gt = sv > thv
            eq = sv == thv
            eqi = jnp.where(eq, 1, 0)
            eqexc = plsc.cumsum(eqi) - eqi
            sel = jnp.logical_or(gt, jnp.logical_and(eq, (eqrun + eqexc) < tie_budget))
            seli = jnp.where(sel, 1, 0)
            dest = jnp.minimum(nsel + plsc.cumsum(seli) - seli, KPAD - 1)
            idxv = lane + 16 * c
            plsc.store_scatter(cs_r, [dest], sv, mask=sel)
            plsc.store_scatter(ci_r, [dest], idxv, mask=sel)
            nsel = nsel + plsc.all_reduce_population_count(sel)
            eqrun = eqrun + plsc.all_reduce_population_count(eq)

        csv = [cs_r[pl.ds(16 * c, 16)] for c in range(KCHUNK)]
        civ = [ci_r[pl.ds(16 * c, 16)] for c in range(KCHUNK)]

        def make_rank_seg(s):
            def rank_body(j, rk):
                sj = vsplat(csv[s], j - 16 * s)
                ij = vsplat(civ[s], j - 16 * s)
                out = []
                for c in range(KCHUNK):
                    win = jnp.logical_or(
                        sj > csv[c],
                        jnp.logical_and(sj == csv[c], ij < civ[c]))
                    out.append(rk[c] + jnp.where(win, 1, 0))
                return tuple(out)
            return rank_body

        init6 = jnp.where(lane >= 4, lane - 4, 0)
        rank = tuple([zeros_i] * (KCHUNK - 1)) + (init6,)
        for s in range(KCHUNK):
            rank = lax.fori_loop(16 * s, min(16 * (s + 1), PRE),
                                 make_rank_seg(s), rank)
        for c in range(KCHUNK):
            plsc.store_scatter(ss_r, [rank[c]], csv[c])
            plsc.store_scatter(pu_r, [rank[c]], civ[c])

        half = jnp.float32(0.5)
        for c in range(KCHUNK):
            gi = jnp.minimum(pu_r[pl.ds(16 * c, 16)], NPAD - 1) + off
            bcx = plsc.load_gather(cxbuf, [gi])
            bcy = plsc.load_gather(cybuf, [gi])
            bw = plsc.load_gather(wbuf, [gi])
            bh = plsc.load_gather(hbuf, [gi])
            x1 = (bcx - half * bw) * swv
            y1 = (bcy - half * bh) * shv
            x2 = (bcx + half * bw) * swv
            y2 = (bcy + half * bh) * shv
            sx1_r[pl.ds(16 * c, 16)] = x1
            sy1_r[pl.ds(16 * c, 16)] = y1
            sx2_r[pl.ds(16 * c, 16)] = x2
            sy2_r[pl.ds(16 * c, 16)] = y2
            ar_r[pl.ds(16 * c, 16)] = (x2 - x1) * (y2 - y1)
            sup_r[pl.ds(16 * c, 16)] = zeros_i

        x1v = [sx1_r[pl.ds(16 * c, 16)] for c in range(KCHUNK)]
        y1v = [sy1_r[pl.ds(16 * c, 16)] for c in range(KCHUNK)]
        x2v = [sx2_r[pl.ds(16 * c, 16)] for c in range(KCHUNK)]
        y2v = [sy2_r[pl.ds(16 * c, 16)] for c in range(KCHUNK)]
        arv = [ar_r[pl.ds(16 * c, 16)] for c in range(KCHUNK)]
        thr = jnp.float32(0.7)
        eps = jnp.float32(1e-9)
        fzero = jnp.float32(0.0)

        def make_seg(s):
            def seg_body(i, sup_s):
                l = i - 16 * s
                x1i = vsplat(x1v[s], l)
                y1i = vsplat(y1v[s], l)
                x2i = vsplat(x2v[s], l)
                y2i = vsplat(y2v[s], l)
                ari = vsplat(arv[s], l)
                actv = vsplat(sup_s, l) == 0
                iv = jnp.full((16,), i, jnp.int32)
                new_sup = sup_s
                for c in range(s, KCHUNK):
                    xx1 = jnp.maximum(x1i, x1v[c])
                    yy1 = jnp.maximum(y1i, y1v[c])
                    xx2 = jnp.minimum(x2i, x2v[c])
                    yy2 = jnp.minimum(y2i, y2v[c])
                    ww = jnp.maximum(xx2 - xx1, fzero)
                    hh = jnp.maximum(yy2 - yy1, fzero)
                    inter = ww * hh
                    union = ari + arv[c] - inter
                    iou = inter / (union + eps)
                    cond = jnp.logical_and(iou > thr, actv)
                    if c == s:
                        cond = jnp.logical_and(cond, (lane + 16 * c) > iv)
                        new_sup = jnp.bitwise_or(new_sup,
                                                 jnp.where(cond, 1, 0))
                    else:
                        supc = sup_r[pl.ds(16 * c, 16)]
                        sup_r[pl.ds(16 * c, 16)] = jnp.bitwise_or(
                            supc, jnp.where(cond, 1, 0))
                return new_sup
            return seg_body

        for s in range(KCHUNK):
            sup_fin = lax.fori_loop(16 * s, min(16 * (s + 1), PRE),
                                    make_seg(s), sup_r[pl.ds(16 * s, 16)])
            sup_r[pl.ds(16 * s, 16)] = sup_fin

        ru = zeros_i
        rv = zeros_i
        for c in range(KCHUNK):
            supc = sup_r[pl.ds(16 * c, 16)]
            alivec = supc == 0
            if c == KCHUNK - 1:
                real = (lane + 16 * c) < PRE
                ub = jnp.logical_and(alivec, real)
                vb = jnp.logical_and(supc != 0, real)
            else:
                ub = alivec
                vb = supc != 0
            u = jnp.where(ub, 1, 0)
            v = jnp.where(vb, 1, 0)
            pu_r[pl.ds(16 * c, 16)] = ru + plsc.cumsum(u) - u
            pv_r[pl.ds(16 * c, 16)] = rv + plsc.cumsum(v) - v
            ru = ru + plsc.all_reduce_population_count(ub)
            rv = rv + plsc.all_reduce_population_count(vb)
        uv = ru
        for c in range(KCHUNK):
            supc = sup_r[pl.ds(16 * c, 16)]
            unsup = supc == 0
            kpos = jnp.where(unsup, pu_r[pl.ds(16 * c, 16)],
                             uv + pv_r[pl.ds(16 * c, 16)])
            if c == KCHUNK - 1:
                m20 = jnp.logical_and(kpos < KEEP, (lane + 16 * c) < PRE)
            else:
                m20 = kpos < KEEP
            kcl = jnp.minimum(kpos, 31)
            plsc.store_scatter(stage_r, [kcl], ss_r[pl.ds(16 * c, 16)], mask=m20)
            plsc.store_scatter(stage_r, [kcl + 32], sx1_r[pl.ds(16 * c, 16)], mask=m20)
            plsc.store_scatter(stage_r, [kcl + 64], sy1_r[pl.ds(16 * c, 16)], mask=m20)
            plsc.store_scatter(stage_r, [kcl + 96], sx2_r[pl.ds(16 * c, 16)], mask=m20)
            plsc.store_scatter(stage_r, [kcl + 128], sy2_r[pl.ds(16 * c, 16)], mask=m20)

        pltpu.sync_copy(stage_r, out_hbm.at[pl.ds((base + k) * OUTW, OUTW)])
        return 0

    lax.fori_loop(0, PER_W, patch_body, 0)


@jax.jit
def kernel(pred_logits, pred_boxes, target_sizes):
    bs, n, _ = pred_logits.shape
    scores = jax.nn.sigmoid(pred_logits[..., -1]).reshape(NPATCH, NQ)
    s_pad = jnp.pad(scores, ((0, 0), (0, NPAD - NQ)), constant_values=-1.0)
    boxes = pred_boxes.reshape(NPATCH, NQ, 4)
    comps = [jnp.pad(boxes[..., i], ((0, 0), (0, NPAD - NQ))) for i in range(4)]

    img_h = target_sizes[:, 0]
    img_w = target_sizes[:, 1]
    sw = jnp.repeat(img_w, NPATCH // bs).reshape(32, 8)
    sh = jnp.repeat(img_h, NPATCH // bs).reshape(32, 8)
    swsh = jnp.concatenate([sw, sh], axis=1).reshape(-1)

    mesh = plsc.VectorSubcoreMesh(core_axis_name="c", subcore_axis_name="s",
                                  num_cores=2, num_subcores=16)
    run = pl.kernel(
        _nms_body,
        out_type=jax.ShapeDtypeStruct((NPATCH * OUTW,), jnp.float32),
        mesh=mesh,
        compiler_params=pltpu.CompilerParams(needs_layout_passes=False),
        scratch_types=[
            pltpu.VMEM((PER_W * NPAD,), jnp.float32),
            pltpu.VMEM((PER_W * NPAD,), jnp.float32),
            pltpu.VMEM((PER_W * NPAD,), jnp.float32),
            pltpu.VMEM((PER_W * NPAD,), jnp.float32),
            pltpu.VMEM((PER_W * NPAD,), jnp.float32),
            pltpu.VMEM((16,), jnp.float32),
            pltpu.VMEM((KPAD,), jnp.float32),
            pltpu.VMEM((KPAD,), jnp.int32),
            pltpu.VMEM((KPAD,), jnp.float32),
            pltpu.VMEM((KPAD,), jnp.float32),
            pltpu.VMEM((KPAD,), jnp.float32),
            pltpu.VMEM((KPAD,), jnp.float32),
            pltpu.VMEM((KPAD,), jnp.float32),
            pltpu.VMEM((KPAD,), jnp.float32),
            pltpu.VMEM((KPAD,), jnp.int32),
            pltpu.VMEM((KPAD,), jnp.int32),
            pltpu.VMEM((KPAD,), jnp.int32),
            pltpu.VMEM((OUTW,), jnp.float32),
        ],
    )
    flat = run(s_pad.reshape(-1), comps[0].reshape(-1), comps[1].reshape(-1),
               comps[2].reshape(-1), comps[3].reshape(-1), swsh)
    out = flat.reshape(NPATCH, 5, 32)[:, :, :KEEP]
    return out.transpose(0, 2, 1).reshape(bs, NPATCH // bs, KEEP, 5)


# speedup vs baseline: 1.2256x; 1.2256x over previous
import functools

import jax
import jax.numpy as jnp
import numpy as np
from jax import lax
from jax.experimental import pallas as pl
from jax.experimental.pallas import tpu as pltpu
from jax.experimental.pallas import tpu_sc as plsc

NQ = 300
NPATCH = 256
NPAD = 304
NCHUNK = NPAD // 16
KPAD = 112
KCHUNK = KPAD // 16
PRE = 100
KEEP = 20
OUTW = 160
PER_W = 8

_LANE = np.arange(16, dtype=np.int32)


def _nms_body(s_hbm, cx_hbm, cy_hbm, w_hbm, h_hbm, swsh_hbm, out_hbm,
              svbuf, cxbuf, cybuf, wbuf, hbuf, swshv,
              cs_r, ci_r, ss_r, sx1_r, sy1_r, sx2_r, sy2_r, ar_r,
              sup_r, pu_r, pv_r, stage_r):
    ncores = 2
    wid = lax.axis_index("s") * ncores + lax.axis_index("c")
    base = wid * PER_W

    pltpu.sync_copy(s_hbm.at[pl.ds(wid * (PER_W * NPAD), PER_W * NPAD)], svbuf)
    pltpu.sync_copy(cx_hbm.at[pl.ds(wid * (PER_W * NPAD), PER_W * NPAD)], cxbuf)
    pltpu.sync_copy(cy_hbm.at[pl.ds(wid * (PER_W * NPAD), PER_W * NPAD)], cybuf)
    pltpu.sync_copy(w_hbm.at[pl.ds(wid * (PER_W * NPAD), PER_W * NPAD)], wbuf)
    pltpu.sync_copy(h_hbm.at[pl.ds(wid * (PER_W * NPAD), PER_W * NPAD)], hbuf)
    pltpu.sync_copy(swsh_hbm.at[pl.ds(wid * 16, 16)], swshv)

    lane = lax.iota(jnp.int32, 16)
    zeros_i = jnp.full((16,), 0, jnp.int32)

    def splat(ref, i):
        return plsc.load_gather(ref, [jnp.full((16,), i, jnp.int32)])

    def vsplat(vec, l):
        return vec.at[jnp.full((16,), l, jnp.int32)].get(
            mode="promise_in_bounds")

    def patch_body(k, _):
        off = k * NPAD
        swv = splat(swshv, k)
        shv = splat(swshv, k + 8)

        svs = [plsc.load_gather(svbuf, [lane + (off + 16 * c)])
               for c in range(NCHUNK)]

        def bit_body(t, kbits):
            bitv = jnp.full((16,), lax.shift_left(jnp.int32(1), 29 - t),
                            jnp.int32)
            trial = jnp.bitwise_or(kbits, bitv)
            tv = plsc.bitcast(trial, jnp.float32)
            cnt = zeros_i
            for c in range(NCHUNK):
                cnt = cnt + plsc.all_reduce_population_count(svs[c] >= tv)
            return jnp.where(cnt >= PRE, trial, kbits)

        kbits = lax.fori_loop(0, 30, bit_body, zeros_i)
        thv = plsc.bitcast(kbits, jnp.float32)

        gcnt = zeros_i
        for c in range(NCHUNK):
            gcnt = gcnt + plsc.all_reduce_population_count(svs[c] > thv)
        tie_budget = PRE - gcnt

        ci_r[pl.ds(96, 16)] = lane + NPAD
        cs_r[pl.ds(96, 16)] = jnp.full((16,), -1.0, jnp.float32)
        nsel = zeros_i
        eqrun = zeros_i
        for c in range(NCHUNK):
            sv = svs[c]
            gt = sv > thv
            eq = sv == thv
            eqi = jnp.where(eq, 1, 0)
            eqexc = plsc.cumsum(eqi) - eqi
            sel = jnp.logical_or(gt, jnp.logical_and(eq, (eqrun + eqexc) < tie_budget))
            seli = jnp.where(sel, 1, 0)
            dest = jnp.minimum(nsel + plsc.cumsum(seli) - seli, KPAD - 1)
            idxv = lane + 16 * c
            plsc.store_scatter(cs_r, [dest], sv, mask=sel)
            plsc.store_scatter(ci_r, [dest], idxv, mask=sel)
            nsel = nsel + plsc.all_reduce_population_count(sel)
            eqrun = eqrun + plsc.all_reduce_population_count(eq)

        csv = [cs_r[pl.ds(16 * c, 16)] for c in range(KCHUNK)]
        civ = [ci_r[pl.ds(16 * c, 16)] for c in range(KCHUNK)]

        def make_rank_seg(s):
            def rank_body(j, rk):
                sj = vsplat(csv[s], j - 16 * s)
                ij = vsplat(civ[s], j - 16 * s)
                out = []
                for c in range(KCHUNK):
                    win = jnp.logical_or(
                        sj > csv[c],
                        jnp.logical_and(sj == csv[c], ij < civ[c]))
                    out.append(rk[c] + jnp.where(win, 1, 0))
                return tuple(out)
            return rank_body

        init6 = jnp.where(lane >= 4, lane - 4, 0)
        rank = tuple([zeros_i] * (KCHUNK - 1)) + (init6,)
        for s in range(KCHUNK):
            rank = lax.fori_loop(16 * s, min(16 * (s + 1), PRE),
                                 make_rank_seg(s), rank)
        for c in range(KCHUNK):
            plsc.store_scatter(ss_r, [rank[c]], csv[c])
            plsc.store_scatter(pu_r, [rank[c]], civ[c])

        half = jnp.float32(0.5)
        for c in range(KCHUNK):
            gi = jnp.minimum(pu_r[pl.ds(16 * c, 16)], NPAD - 1) + off
            bcx = plsc.load_gather(cxbuf, [gi])
            bcy = plsc.load_gather(cybuf, [gi])
            bw = plsc.load_gather(wbuf, [gi])
            bh = plsc.load_gather(hbuf, [gi])
            x1 = (bcx - half * bw) * swv
            y1 = (bcy - half * bh) * shv
            x2 = (bcx + half * bw) * swv
            y2 = (bcy + half * bh) * shv
            sx1_r[pl.ds(16 * c, 16)] = x1
            sy1_r[pl.ds(16 * c, 16)] = y1
            sx2_r[pl.ds(16 * c, 16)] = x2
            sy2_r[pl.ds(16 * c, 16)] = y2
            ar_r[pl.ds(16 * c, 16)] = (x2 - x1) * (y2 - y1)
            sup_r[pl.ds(16 * c, 16)] = zeros_i

        x1v = [sx1_r[pl.ds(16 * c, 16)] for c in range(KCHUNK)]
        y1v = [sy1_r[pl.ds(16 * c, 16)] for c in range(KCHUNK)]
        x2v = [sx2_r[pl.ds(16 * c, 16)] for c in range(KCHUNK)]
        y2v = [sy2_r[pl.ds(16 * c, 16)] for c in range(KCHUNK)]
        arv = [ar_r[pl.ds(16 * c, 16)] for c in range(KCHUNK)]
        thr = jnp.float32(0.7)
        eps = jnp.float32(1e-9)
        fzero = jnp.float32(0.0)

        def make_seg(s, seg_end):
            def seg_cond(carry):
                i, kept = carry
                return jnp.logical_and(i < seg_end, kept < KEEP)

            def seg_body(carry):
                i, kept = carry
                l = i - 16 * s
                sup_s = sup_r[pl.ds(16 * s, 16)]
                x1i = vsplat(x1v[s], l)
                y1i = vsplat(y1v[s], l)
                x2i = vsplat(x2v[s], l)
                y2i = vsplat(y2v[s], l)
                ari = vsplat(arv[s], l)
                sup_splat = vsplat(sup_s, l)
                sup_scal = sup_splat[0]
                actv = sup_splat == 0
                iv = jnp.full((16,), i, jnp.int32)
                for c in range(s, KCHUNK):
                    xx1 = jnp.maximum(x1i, x1v[c])
                    yy1 = jnp.maximum(y1i, y1v[c])
                    xx2 = jnp.minimum(x2i, x2v[c])
                    yy2 = jnp.minimum(y2i, y2v[c])
                    ww = jnp.maximum(xx2 - xx1, fzero)
                    hh = jnp.maximum(yy2 - yy1, fzero)
                    inter = ww * hh
                    union = ari + arv[c] - inter
                    iou = inter / (union + eps)
                    cond = jnp.logical_and(iou > thr, actv)
                    if c == s:
                        cond = jnp.logical_and(cond, (lane + 16 * c) > iv)
                        sup_r[pl.ds(16 * c, 16)] = jnp.bitwise_or(
                            sup_s, jnp.where(cond, 1, 0))
                    else:
                        supc = sup_r[pl.ds(16 * c, 16)]
                        sup_r[pl.ds(16 * c, 16)] = jnp.bitwise_or(
                            supc, jnp.where(cond, 1, 0))
                return i + 1, kept + (1 - sup_scal)
            return seg_cond, seg_body

        kept = jnp.int32(0)
        for s in range(KCHUNK):
            seg_cond, seg_body = make_seg(s, min(16 * (s + 1), PRE))
            _, kept = lax.while_loop(seg_cond, seg_body,
                                     (jnp.int32(16 * s), kept))

        ru = zeros_i
        rv = zeros_i
        for c in range(KCHUNK):
            supc = sup_r[pl.ds(16 * c, 16)]
            alivec = supc == 0
            if c == KCHUNK - 1:
                real = (lane + 16 * c) < PRE
                ub = jnp.logical_and(alivec, real)
                vb = jnp.logical_and(supc != 0, real)
            else:
                ub = alivec
                vb = supc != 0
            u = jnp.where(ub, 1, 0)
            v = jnp.where(vb, 1, 0)
            pu_r[pl.ds(16 * c, 16)] = ru + plsc.cumsum(u) - u
            pv_r[pl.ds(16 * c, 16)] = rv + plsc.cumsum(v) - v
            ru = ru + plsc.all_reduce_population_count(ub)
            rv = rv + plsc.all_reduce_population_count(vb)
        uv = ru
        for c in range(KCHUNK):
            supc = sup_r[pl.ds(16 * c, 16)]
            unsup = supc == 0
            kpos = jnp.where(unsup, pu_r[pl.ds(16 * c, 16)],
                             uv + pv_r[pl.ds(16 * c, 16)])
            if c == KCHUNK - 1:
                m20 = jnp.logical_and(kpos < KEEP, (lane + 16 * c) < PRE)
            else:
                m20 = kpos < KEEP
            kcl = jnp.minimum(kpos, 31)
            plsc.store_scatter(stage_r, [kcl], ss_r[pl.ds(16 * c, 16)], mask=m20)
            plsc.store_scatter(stage_r, [kcl + 32], sx1_r[pl.ds(16 * c, 16)], mask=m20)
            plsc.store_scatter(stage_r, [kcl + 64], sy1_r[pl.ds(16 * c, 16)], mask=m20)
            plsc.store_scatter(stage_r, [kcl + 96], sx2_r[pl.ds(16 * c, 16)], mask=m20)
            plsc.store_scatter(stage_r, [kcl + 128], sy2_r[pl.ds(16 * c, 16)], mask=m20)

        pltpu.sync_copy(stage_r, out_hbm.at[pl.ds((base + k) * OUTW, OUTW)])
        return 0

    lax.fori_loop(0, PER_W, patch_body, 0)


@jax.jit
def kernel(pred_logits, pred_boxes, target_sizes):
    bs, n, _ = pred_logits.shape
    scores = jax.nn.sigmoid(pred_logits[..., -1]).reshape(NPATCH, NQ)
    s_pad = jnp.pad(scores, ((0, 0), (0, NPAD - NQ)), constant_values=-1.0)
    boxes = pred_boxes.reshape(NPATCH, NQ, 4)
    comps = [jnp.pad(boxes[..., i], ((0, 0), (0, NPAD - NQ))) for i in range(4)]

    img_h = target_sizes[:, 0]
    img_w = target_sizes[:, 1]
    sw = jnp.repeat(img_w, NPATCH // bs).reshape(32, 8)
    sh = jnp.repeat(img_h, NPATCH // bs).reshape(32, 8)
    swsh = jnp.concatenate([sw, sh], axis=1).reshape(-1)

    mesh = plsc.VectorSubcoreMesh(core_axis_name="c", subcore_axis_name="s",
                                  num_cores=2, num_subcores=16)
    run = pl.kernel(
        _nms_body,
        out_type=jax.ShapeDtypeStruct((NPATCH * OUTW,), jnp.float32),
        mesh=mesh,
        compiler_params=pltpu.CompilerParams(needs_layout_passes=False),
        scratch_types=[
            pltpu.VMEM((PER_W * NPAD,), jnp.float32),
            pltpu.VMEM((PER_W * NPAD,), jnp.float32),
            pltpu.VMEM((PER_W * NPAD,), jnp.float32),
            pltpu.VMEM((PER_W * NPAD,), jnp.float32),
            pltpu.VMEM((PER_W * NPAD,), jnp.float32),
            pltpu.VMEM((16,), jnp.float32),
            pltpu.VMEM((KPAD,), jnp.float32),
            pltpu.VMEM((KPAD,), jnp.int32),
            pltpu.VMEM((KPAD,), jnp.float32),
            pltpu.VMEM((KPAD,), jnp.float32),
            pltpu.VMEM((KPAD,), jnp.float32),
            pltpu.VMEM((KPAD,), jnp.float32),
            pltpu.VMEM((KPAD,), jnp.float32),
            pltpu.VMEM((KPAD,), jnp.float32),
            pltpu.VMEM((KPAD,), jnp.int32),
            pltpu.VMEM((KPAD,), jnp.int32),
            pltpu.VMEM((KPAD,), jnp.int32),
            pltpu.VMEM((OUTW,), jnp.float32),
        ],
    )
    flat = run(s_pad.reshape(-1), comps[0].reshape(-1), comps[1].reshape(-1),
               comps[2].reshape(-1), comps[3].reshape(-1), swsh)
    out = flat.reshape(NPATCH, 5, 32)[:, :, :KEEP]
    return out.transpose(0, 2, 1).reshape(bs, NPATCH // bs, KEEP, 5)


# speedup vs baseline: 1.5053x; 1.2281x over previous
import functools

import jax
import jax.numpy as jnp
import numpy as np
from jax import lax
from jax.experimental import pallas as pl
from jax.experimental.pallas import tpu as pltpu
from jax.experimental.pallas import tpu_sc as plsc

NQ = 300
NPATCH = 256
NPAD = 304
NCHUNK = NPAD // 16
KPAD = 112
KCHUNK = KPAD // 16
PRE = 100
FAST = 32
FCHUNK = FAST // 16
KEEP = 20
OUTW = 160
PER_W = 8

_LANE = np.arange(16, dtype=np.int32)


def _nms_body(s_hbm, cx_hbm, cy_hbm, w_hbm, h_hbm, swsh_hbm, out_hbm,
              svbuf, cxbuf, cybuf, wbuf, hbuf, swshv,
              cs_r, ci_r, ss_r, sx1_r, sy1_r, sx2_r, sy2_r, ar_r,
              sup_r, pu_r, pv_r, stage_r):
    ncores = 2
    wid = lax.axis_index("s") * ncores + lax.axis_index("c")
    base = wid * PER_W

    pltpu.sync_copy(s_hbm.at[pl.ds(wid * (PER_W * NPAD), PER_W * NPAD)], svbuf)
    pltpu.sync_copy(cx_hbm.at[pl.ds(wid * (PER_W * NPAD), PER_W * NPAD)], cxbuf)
    pltpu.sync_copy(cy_hbm.at[pl.ds(wid * (PER_W * NPAD), PER_W * NPAD)], cybuf)
    pltpu.sync_copy(w_hbm.at[pl.ds(wid * (PER_W * NPAD), PER_W * NPAD)], wbuf)
    pltpu.sync_copy(h_hbm.at[pl.ds(wid * (PER_W * NPAD), PER_W * NPAD)], hbuf)
    pltpu.sync_copy(swsh_hbm.at[pl.ds(wid * 16, 16)], swshv)

    lane = lax.iota(jnp.int32, 16)
    zeros_i = jnp.full((16,), 0, jnp.int32)

    def splat(ref, i):
        return plsc.load_gather(ref, [jnp.full((16,), i, jnp.int32)])

    def vsplat(vec, l):
        return vec.at[jnp.full((16,), l, jnp.int32)].get(
            mode="promise_in_bounds")

    def patch_body(k, _):
        off = k * NPAD
        swv = splat(swshv, k)
        shv = splat(swshv, k + 8)

        svs = [plsc.load_gather(svbuf, [lane + (off + 16 * c)])
               for c in range(NCHUNK)]

        def bitsearch(target):
            def bit_body(t, kbits):
                bitv = jnp.full((16,), lax.shift_left(jnp.int32(1), 29 - t),
                                jnp.int32)
                trial = jnp.bitwise_or(kbits, bitv)
                tv = plsc.bitcast(trial, jnp.float32)
                cnt = zeros_i
                for c in range(NCHUNK):
                    cnt = cnt + plsc.all_reduce_population_count(svs[c] >= tv)
                return jnp.where(cnt >= target, trial, kbits)

            kbits = lax.fori_loop(0, 30, bit_body, zeros_i)
            thv = plsc.bitcast(kbits, jnp.float32)
            gcnt = zeros_i
            for c in range(NCHUNK):
                gcnt = gcnt + plsc.all_reduce_population_count(svs[c] > thv)
            return thv, target - gcnt

        def compact(thv, tie_budget):
            nsel = zeros_i
            eqrun = zeros_i
            for c in range(NCHUNK):
                sv = svs[c]
                gt = sv > thv
                eq = sv == thv
                eqi = jnp.where(eq, 1, 0)
                eqexc = plsc.cumsum(eqi) - eqi
                sel = jnp.logical_or(
                    gt, jnp.logical_and(eq, (eqrun + eqexc) < tie_budget))
                seli = jnp.where(sel, 1, 0)
                dest = jnp.minimum(nsel + plsc.cumsum(seli) - seli, KPAD - 1)
                idxv = lane + 16 * c
                plsc.store_scatter(cs_r, [dest], sv, mask=sel)
                plsc.store_scatter(ci_r, [dest], idxv, mask=sel)
                nsel = nsel + plsc.all_reduce_population_count(sel)
                eqrun = eqrun + plsc.all_reduce_population_count(eq)

        def ranksort(nc, npre, pads):
            csv = [cs_r[pl.ds(16 * c, 16)] for c in range(nc)]
            civ = [ci_r[pl.ds(16 * c, 16)] for c in range(nc)]

            def make_rank_seg(s):
                def rank_body(j, rk):
                    sj = vsplat(csv[s], j - 16 * s)
                    ij = vsplat(civ[s], j - 16 * s)
                    out = []
                    for c in range(nc):
                        win = jnp.logical_or(
                            sj > csv[c],
                            jnp.logical_and(sj == csv[c], ij < civ[c]))
                        out.append(rk[c] + jnp.where(win, 1, 0))
                    return tuple(out)
                return rank_body

            npad0 = npre - 16 * (nc - 1)
            if pads:
                init_last = jnp.where(lane >= npad0, lane - npad0, 0)
            else:
                init_last = zeros_i
            rank = tuple([zeros_i] * (nc - 1)) + (init_last,)
            for s in range(nc):
                rank = lax.fori_loop(16 * s, min(16 * (s + 1), npre),
                                     make_rank_seg(s), rank)
            for c in range(nc):
                plsc.store_scatter(ss_r, [rank[c]], csv[c])
                plsc.store_scatter(pu_r, [rank[c]], civ[c])

        def gather_boxes(nc):
            half = jnp.float32(0.5)
            for c in range(nc):
                gi = jnp.minimum(pu_r[pl.ds(16 * c, 16)], NPAD - 1) + off
                bcx = plsc.load_gather(cxbuf, [gi])
                bcy = plsc.load_gather(cybuf, [gi])
                bw = plsc.load_gather(wbuf, [gi])
                bh = plsc.load_gather(hbuf, [gi])
                x1 = (bcx - half * bw) * swv
                y1 = (bcy - half * bh) * shv
                x2 = (bcx + half * bw) * swv
                y2 = (bcy + half * bh) * shv
                sx1_r[pl.ds(16 * c, 16)] = x1
                sy1_r[pl.ds(16 * c, 16)] = y1
                sx2_r[pl.ds(16 * c, 16)] = x2
                sy2_r[pl.ds(16 * c, 16)] = y2
                ar_r[pl.ds(16 * c, 16)] = (x2 - x1) * (y2 - y1)
                sup_r[pl.ds(16 * c, 16)] = zeros_i

        def nms(nc, npre):
            x1v = [sx1_r[pl.ds(16 * c, 16)] for c in range(nc)]
            y1v = [sy1_r[pl.ds(16 * c, 16)] for c in range(nc)]
            x2v = [sx2_r[pl.ds(16 * c, 16)] for c in range(nc)]
            y2v = [sy2_r[pl.ds(16 * c, 16)] for c in range(nc)]
            arv = [ar_r[pl.ds(16 * c, 16)] for c in range(nc)]
            thr = jnp.float32(0.7)
            eps = jnp.float32(1e-9)
            fzero = jnp.float32(0.0)

            def make_seg(s, seg_end):
                def seg_cond(carry):
                    i, kept = carry
                    return jnp.logical_and(i < seg_end, kept < KEEP)

                def seg_body(carry):
                    i, kept = carry
                    l = i - 16 * s
                    sup_s = sup_r[pl.ds(16 * s, 16)]
                    x1i = vsplat(x1v[s], l)
                    y1i = vsplat(y1v[s], l)
                    x2i = vsplat(x2v[s], l)
                    y2i = vsplat(y2v[s], l)
                    ari = vsplat(arv[s], l)
                    sup_splat = vsplat(sup_s, l)
                    sup_scal = sup_splat[0]
                    actv = sup_splat == 0
                    iv = jnp.full((16,), i, jnp.int32)
                    for c in range(s, nc):
                        xx1 = jnp.maximum(x1i, x1v[c])
                        yy1 = jnp.maximum(y1i, y1v[c])
                        xx2 = jnp.minimum(x2i, x2v[c])
                        yy2 = jnp.minimum(y2i, y2v[c])
                        ww = jnp.maximum(xx2 - xx1, fzero)
                        hh = jnp.maximum(yy2 - yy1, fzero)
                        inter = ww * hh
                        union = ari + arv[c] - inter
                        iou = inter / (union + eps)
                        cond = jnp.logical_and(iou > thr, actv)
                        if c == s:
                            cond = jnp.logical_and(cond, (lane + 16 * c) > iv)
                            sup_r[pl.ds(16 * c, 16)] = jnp.bitwise_or(
                                sup_s, jnp.where(cond, 1, 0))
                        else:
                            supc = sup_r[pl.ds(16 * c, 16)]
                            sup_r[pl.ds(16 * c, 16)] = jnp.bitwise_or(
                                supc, jnp.where(cond, 1, 0))
                    return i + 1, kept + (1 - sup_scal)
                return seg_cond, seg_body

            kept = jnp.int32(0)
            for s in range(nc):
                seg_cond, seg_body = make_seg(s, min(16 * (s + 1), npre))
                _, kept = lax.while_loop(seg_cond, seg_body,
                                         (jnp.int32(16 * s), kept))
            return kept

        def write_rows(c, kcl, m20):
            plsc.store_scatter(stage_r, [kcl], ss_r[pl.ds(16 * c, 16)],
                               mask=m20)
            plsc.store_scatter(stage_r, [kcl + 32], sx1_r[pl.ds(16 * c, 16)],
                               mask=m20)
            plsc.store_scatter(stage_r, [kcl + 64], sy1_r[pl.ds(16 * c, 16)],
                               mask=m20)
            plsc.store_scatter(stage_r, [kcl + 96], sx2_r[pl.ds(16 * c, 16)],
                               mask=m20)
            plsc.store_scatter(stage_r, [kcl + 128], sy2_r[pl.ds(16 * c, 16)],
                               mask=m20)

        def stage6_fast():
            ru = zeros_i
            for c in range(FCHUNK):
                supc = sup_r[pl.ds(16 * c, 16)]
                alive = supc == 0
                u = jnp.where(alive, 1, 0)
                pu = ru + plsc.cumsum(u) - u
                ru = ru + plsc.all_reduce_population_count(alive)
                m20 = jnp.logical_and(alive, pu < KEEP)
                kcl = jnp.minimum(jnp.where(alive, pu, 31), 31)
                write_rows(c, kcl, m20)

        def stage6_full():
            ru = zeros_i
            rv = zeros_i
            for c in range(KCHUNK):
                supc = sup_r[pl.ds(16 * c, 16)]
                alivec = supc == 0
                if c == KCHUNK - 1:
                    real = (lane + 16 * c) < PRE
                    ub = jnp.logical_and(alivec, real)
                    vb = jnp.logical_and(supc != 0, real)
                else:
                    ub = alivec
                    vb = supc != 0
                u = jnp.where(ub, 1, 0)
                v = jnp.where(vb, 1, 0)
                pu_r[pl.ds(16 * c, 16)] = ru + plsc.cumsum(u) - u
                pv_r[pl.ds(16 * c, 16)] = rv + plsc.cumsum(v) - v
                ru = ru + plsc.all_reduce_population_count(ub)
                rv = rv + plsc.all_reduce_population_count(vb)
            uv = ru
            for c in range(KCHUNK):
                supc = sup_r[pl.ds(16 * c, 16)]
                unsup = supc == 0
                kpos = jnp.where(unsup, pu_r[pl.ds(16 * c, 16)],
                                 uv + pv_r[pl.ds(16 * c, 16)])
                if c == KCHUNK - 1:
                    m20 = jnp.logical_and(kpos < KEEP, (lane + 16 * c) < PRE)
                else:
                    m20 = kpos < KEEP
                kcl = jnp.minimum(kpos, 31)
                write_rows(c, kcl, m20)

        th32, tb32 = bitsearch(FAST)
        compact(th32, tb32)
        ranksort(FCHUNK, FAST, False)
        gather_boxes(FCHUNK)
        kept32 = nms(FCHUNK, FAST)

        def slow_path():
            th, tb = bitsearch(PRE)
            ci_r[pl.ds(96, 16)] = lane + NPAD
            cs_r[pl.ds(96, 16)] = jnp.full((16,), -1.0, jnp.float32)
            compact(th, tb)
            ranksort(KCHUNK, PRE, True)
            gather_boxes(KCHUNK)
            nms(KCHUNK, PRE)
            stage6_full()

        lax.cond(kept32 >= KEEP, stage6_fast, slow_path)

        pltpu.sync_copy(stage_r, out_hbm.at[pl.ds((base + k) * OUTW, OUTW)])
        return 0

    lax.fori_loop(0, PER_W, patch_body, 0)


@jax.jit
def kernel(pred_logits, pred_boxes, target_sizes):
    bs, n, _ = pred_logits.shape
    scores = jax.nn.sigmoid(pred_logits[..., -1]).reshape(NPATCH, NQ)
    s_pad = jnp.pad(scores, ((0, 0), (0, NPAD - NQ)), constant_values=-1.0)
    boxes = pred_boxes.reshape(NPATCH, NQ, 4)
    comps = [jnp.pad(boxes[..., i], ((0, 0), (0, NPAD - NQ))) for i in range(4)]

    img_h = target_sizes[:, 0]
    img_w = target_sizes[:, 1]
    sw = jnp.repeat(img_w, NPATCH // bs).reshape(32, 8)
    sh = jnp.repeat(img_h, NPATCH // bs).reshape(32, 8)
    swsh = jnp.concatenate([sw, sh], axis=1).reshape(-1)

    mesh = plsc.VectorSubcoreMesh(core_axis_name="c", subcore_axis_name="s",
                                  num_cores=2, num_subcores=16)
    run = pl.kernel(
        _nms_body,
        out_type=jax.ShapeDtypeStruct((NPATCH * OUTW,), jnp.float32),
        mesh=mesh,
        compiler_params=pltpu.CompilerParams(needs_layout_passes=False),
        scratch_types=[
            pltpu.VMEM((PER_W * NPAD,), jnp.float32),
            pltpu.VMEM((PER_W * NPAD,), jnp.float32),
            pltpu.VMEM((PER_W * NPAD,), jnp.float32),
            pltpu.VMEM((PER_W * NPAD,), jnp.float32),
            pltpu.VMEM((PER_W * NPAD,), jnp.float32),
            pltpu.VMEM((16,), jnp.float32),
            pltpu.VMEM((KPAD,), jnp.float32),
            pltpu.VMEM((KPAD,), jnp.int32),
            pltpu.VMEM((KPAD,), jnp.float32),
            pltpu.VMEM((KPAD,), jnp.float32),
            pltpu.VMEM((KPAD,), jnp.float32),
            pltpu.VMEM((KPAD,), jnp.float32),
            pltpu.VMEM((KPAD,), jnp.float32),
            pltpu.VMEM((KPAD,), jnp.float32),
            pltpu.VMEM((KPAD,), jnp.int32),
            pltpu.VMEM((KPAD,), jnp.int32),
            pltpu.VMEM((KPAD,), jnp.int32),
            pltpu.VMEM((OUTW,), jnp.float32),
        ],
    )
    flat = run(s_pad.reshape(-1), comps[0].reshape(-1), comps[1].reshape(-1),
               comps[2].reshape(-1), comps[3].reshape(-1), swsh)
    out = flat.reshape(NPATCH, 5, 32)[:, :, :KEEP]
    return out.transpose(0, 2, 1).reshape(bs, NPATCH // bs, KEEP, 5)
